# baseline (speedup 1.0000x reference)
import jax
import jax.numpy as jnp
from jax.experimental import pallas as pl
from jax.experimental.pallas import tpu as pltpu


def _vq_body(z_ref, emb_m2_ref, enorm_ref, emb_b16_ref, embsum_m2_ref,
             enorm_total_ref, zq_ref, idx_ref, perp_ref, mdist_ref,
             counts_ref, dsum_ref):
    b = pl.program_id(0)
    nb = pl.num_programs(0)
    nsl = z_ref.shape[0]
    K, C = emb_m2_ref.shape
    HW = z_ref.shape[2] * z_ref.shape[3]

    cnt = jnp.zeros((1, K), jnp.float32)
    bsum = jnp.float32(0.0)
    ones = jnp.ones((1, HW), jnp.bfloat16)
    for sl in range(nsl):
        zb = z_ref[sl].reshape(C, HW)
        d0 = jax.lax.dot_general(emb_m2_ref[...], zb, (((1,), (0,)), ((), ())),
                                 preferred_element_type=jnp.float32)
        znorm = jnp.sum(zb * zb, axis=0, keepdims=True)
        dist = (znorm + d0) + enorm_ref[...]

        idx = jnp.argmin(dist, axis=0)
        idx_ref[sl, 0, :] = idx

        onehot = (jax.lax.broadcasted_iota(jnp.int16, (K, HW), 0)
                  == idx.astype(jnp.int16)[None, :]).astype(jnp.bfloat16)
        zq = jax.lax.dot_general(emb_b16_ref[...], onehot,
                                 (((0,), (0,)), ((), ())),
                                 preferred_element_type=jnp.float32)
        zq_ref[sl] = zq.reshape(zq_ref.shape[1:])

        cnt = cnt + jax.lax.dot_general(ones, onehot, (((1,), (1,)), ((), ())),
                                        preferred_element_type=jnp.float32)

        ssum = jax.lax.dot_general(embsum_m2_ref[...], zb,
                                   (((1,), (0,)), ((), ())),
                                   preferred_element_type=jnp.float32)
        bsum = bsum + (jnp.float32(K) * jnp.sum(znorm) + jnp.sum(ssum))

    @pl.when(b == 0)
    def _init():
        counts_ref[...] = cnt
        dsum_ref[0, 0] = bsum

    @pl.when(b != 0)
    def _acc():
        counts_ref[...] = counts_ref[...] + cnt
        dsum_ref[0, 0] = dsum_ref[0, 0] + bsum

    @pl.when(b == nb - 1)
    def _finalize():
        n_total = jnp.float32(nb * nsl * HW)
        e_mean = counts_ref[...] / n_total
        perp = jnp.exp(-jnp.sum(e_mean * jnp.log(e_mean + 1e-10)))
        perp_ref[0, 0] = perp
        mdist_ref[0, 0] = ((dsum_ref[0, 0] + n_total * enorm_total_ref[0, 0])
                           / (n_total * jnp.float32(K)))


def kernel(z, embedding):
    b, c, h, w = z.shape
    K = embedding.shape[0]
    hw = h * w

    emb_m2 = -2.0 * embedding
    enorm = jnp.sum(embedding ** 2, axis=1, keepdims=True)
    emb_b16 = embedding.astype(jnp.bfloat16)
    embsum_m2 = jnp.sum(emb_m2, axis=0, keepdims=True)
    enorm_total = jnp.sum(enorm).reshape(1, 1)

    nsl = 2
    grid = (b // nsl,)
    zq3, idx3, perp, mdist = pl.pallas_call(
        _vq_body,
        grid=grid,
        in_specs=[
            pl.BlockSpec((nsl, c, h, w), lambda i: (i, 0, 0, 0)),
            pl.BlockSpec((K, c), lambda i: (0, 0)),
            pl.BlockSpec((K, 1), lambda i: (0, 0)),
            pl.BlockSpec((K, c), lambda i: (0, 0)),
            pl.BlockSpec((1, c), lambda i: (0, 0)),
            pl.BlockSpec(memory_space=pltpu.SMEM),
        ],
        out_specs=[
            pl.BlockSpec((nsl, c, h, w), lambda i: (i, 0, 0, 0)),
            pl.BlockSpec((nsl, 1, hw), lambda i: (i, 0, 0)),
            pl.BlockSpec(memory_space=pltpu.SMEM),
            pl.BlockSpec(memory_space=pltpu.SMEM),
        ],
        out_shape=[
            jax.ShapeDtypeStruct((b, c, h, w), jnp.float32),
            jax.ShapeDtypeStruct((b, 1, hw), jnp.int32),
            jax.ShapeDtypeStruct((1, 1), jnp.float32),
            jax.ShapeDtypeStruct((1, 1), jnp.float32),
        ],
        scratch_shapes=[
            pltpu.VMEM((1, K), jnp.float32),
            pltpu.SMEM((1, 1), jnp.float32),
        ],
    )(z, emb_m2, enorm, emb_b16, embsum_m2, enorm_total)

    z_q = zq3
    indices = idx3.reshape(b, h, w)
    loss = jnp.zeros((), z.dtype)
    return (z_q, loss, perp.reshape(()), indices, mdist.reshape(()))


# speedup vs baseline: 2.3356x; 2.3356x over previous
import jax
import jax.numpy as jnp
from jax.experimental import pallas as pl
from jax.experimental.pallas import tpu as pltpu


def _vq_body(z_ref, emb_m2_ref, enorm_ref, emb_b16_ref, embsum_m2_ref,
             enorm_total_ref, zq_ref, idx_ref, perp_ref, mdist_ref,
             counts_ref, dsum_ref):
    b = pl.program_id(0)
    nb = pl.num_programs(0)
    nsl = z_ref.shape[0]
    K, C = emb_m2_ref.shape
    HW = z_ref.shape[2]

    cnt = jnp.zeros((1, K), jnp.float32)
    bsum = jnp.float32(0.0)
    ones = jnp.ones((1, HW), jnp.bfloat16)
    for sl in range(nsl):
        zb = z_ref[sl]
        d0 = jax.lax.dot_general(emb_m2_ref[...], zb, (((1,), (0,)), ((), ())),
                                 preferred_element_type=jnp.float32)
        znorm = jnp.sum(zb * zb, axis=0, keepdims=True)
        dist = (znorm + d0) + enorm_ref[...]

        idx = jnp.argmin(dist, axis=0)
        idx_ref[sl, 0, :] = idx

        onehot = (jax.lax.broadcasted_iota(jnp.int32, (K, HW), 0)
                  == idx[None, :]).astype(jnp.bfloat16)
        zq = jax.lax.dot_general(emb_b16_ref[...], onehot,
                                 (((0,), (0,)), ((), ())),
                                 preferred_element_type=jnp.float32)
        zq_ref[sl] = zq

        cnt = cnt + jax.lax.dot_general(ones, onehot, (((1,), (1,)), ((), ())),
                                        preferred_element_type=jnp.float32)

        ssum = jax.lax.dot_general(embsum_m2_ref[...], zb,
                                   (((1,), (0,)), ((), ())),
                                   preferred_element_type=jnp.float32)
        bsum = bsum + (jnp.float32(K) * jnp.sum(znorm) + jnp.sum(ssum))

    @pl.when(b == 0)
    def _init():
        counts_ref[...] = cnt
        dsum_ref[0, 0] = bsum

    @pl.when(b != 0)
    def _acc():
        counts_ref[...] = counts_ref[...] + cnt
        dsum_ref[0, 0] = dsum_ref[0, 0] + bsum

    @pl.when(b == nb - 1)
    def _finalize():
        n_total = jnp.float32(nb * nsl * HW)
        e_mean = counts_ref[...] / n_total
        perp = jnp.exp(-jnp.sum(e_mean * jnp.log(e_mean + 1e-10)))
        perp_ref[0, 0] = perp
        mdist_ref[0, 0] = ((dsum_ref[0, 0] + n_total * enorm_total_ref[0, 0])
                           / (n_total * jnp.float32(K)))


def kernel(z, embedding):
    b, c, h, w = z.shape
    K = embedding.shape[0]
    hw = h * w
    z3 = z.reshape(b, c, hw)

    emb_m2 = -2.0 * embedding
    enorm = jnp.sum(embedding ** 2, axis=1, keepdims=True)
    emb_b16 = embedding.astype(jnp.bfloat16)
    embsum_m2 = jnp.sum(emb_m2, axis=0, keepdims=True)
    enorm_total = jnp.sum(enorm).reshape(1, 1)

    nsl = 2
    grid = (b // nsl,)
    zq3, idx3, perp, mdist = pl.pallas_call(
        _vq_body,
        grid=grid,
        in_specs=[
            pl.BlockSpec((nsl, c, hw), lambda i: (i, 0, 0)),
            pl.BlockSpec((K, c), lambda i: (0, 0)),
            pl.BlockSpec((K, 1), lambda i: (0, 0)),
            pl.BlockSpec((K, c), lambda i: (0, 0)),
            pl.BlockSpec((1, c), lambda i: (0, 0)),
            pl.BlockSpec(memory_space=pltpu.SMEM),
        ],
        out_specs=[
            pl.BlockSpec((nsl, c, hw), lambda i: (i, 0, 0)),
            pl.BlockSpec((nsl, 1, hw), lambda i: (i, 0, 0)),
            pl.BlockSpec(memory_space=pltpu.SMEM),
            pl.BlockSpec(memory_space=pltpu.SMEM),
        ],
        out_shape=[
            jax.ShapeDtypeStruct((b, c, hw), jnp.float32),
            jax.ShapeDtypeStruct((b, 1, hw), jnp.int32),
            jax.ShapeDtypeStruct((1, 1), jnp.float32),
            jax.ShapeDtypeStruct((1, 1), jnp.float32),
        ],
        scratch_shapes=[
            pltpu.VMEM((1, K), jnp.float32),
            pltpu.SMEM((1, 1), jnp.float32),
        ],
    )(z3, emb_m2, enorm, emb_b16, embsum_m2, enorm_total)

    z_q = zq3.reshape(b, c, h, w)
    indices = idx3.reshape(b, h, w)
    loss = jnp.zeros((), z.dtype)
    return (z_q, loss, perp.reshape(()), indices, mdist.reshape(()))
